# baseline (speedup 1.0000x reference)
import functools

import jax
import jax.numpy as jnp
from jax import lax
from jax.experimental import pallas as pl
from jax.experimental.pallas import tpu as pltpu
from jax.experimental.pallas import tpu_sc as plsc

_N = 10000
_DEG = 32
_E = _N * _DEG
_H = 128
_K = 4
_D2 = _DEG * 64

_NC = 2
_NS = 16
_NW = _NC * _NS
_EW = _E // _NW
_C = 80
_J = _EW // _C
_G = 5
_NG = _J // _G

_NB = 200
_GRID = _N // _NB


def _mlp_front(h, W1T, b1):
    def body(h_ref, w_ref, b1_ref, a_out, b_out):
        hh = h_ref[...]
        w = w_ref[...]
        a_out[...] = lax.dot_general(
            hh, w[:_H], (((1,), (0,)), ((), ())),
            preferred_element_type=jnp.float32) + b1_ref[...]
        b_out[...] = lax.dot_general(
            hh, w[_H:], (((1,), (0,)), ((), ())),
            preferred_element_type=jnp.float32)

    return pl.pallas_call(
        body,
        out_shape=[
            jax.ShapeDtypeStruct((_N, 64), jnp.float32),
            jax.ShapeDtypeStruct((_N, 64), jnp.float32),
        ],
    )(h, W1T, b1)


def _sc_gather(B, dst3):
    mesh = plsc.VectorSubcoreMesh(core_axis_name="c", subcore_axis_name="s")

    @functools.partial(
        pl.kernel,
        out_type=jax.ShapeDtypeStruct((_E, 64), jnp.float32),
        mesh=mesh,
        compiler_params=pltpu.CompilerParams(use_tc_tiling_on_sc=False),
        scratch_types=[
            pltpu.VMEM((_J, _C), jnp.int32),
            [pltpu.VMEM((_C, 64), jnp.float32) for _ in range(2 * _G)],
            pltpu.SemaphoreType.DMA,
            pltpu.SemaphoreType.DMA,
            pltpu.SemaphoreType.DMA,
            pltpu.SemaphoreType.DMA,
        ],
    )
    def k(b_hbm, dst_hbm, out_hbm, idx_v, bufs, sga, sgb, ssa, ssb):
        wid = lax.axis_index("s") * _NC + lax.axis_index("c")
        pltpu.sync_copy(dst_hbm.at[wid], idx_v)
        ebase = wid * _EW
        seta, setb = bufs[:_G], bufs[_G:]

        def fire_gathers(g, bset, sem):
            return [
                pltpu.async_copy(b_hbm.at[idx_v.at[g * _G + b]], bset[b], sem)
                for b in range(_G)
            ]

        def out_slice(g, b):
            off = pl.multiple_of(ebase + (g * _G + b) * _C, 8)
            return out_hbm.at[pl.ds(off, _C)]

        def fire_stores(g, bset, sem):
            for b in range(_G):
                pltpu.async_copy(bset[b], out_slice(g, b), sem)

        def drain_stores(g, bset, sem):
            for b in range(_G):
                pltpu.make_async_copy(bset[b], out_slice(g, b), sem).wait()

        def run_group(g, bset, sem_g, sem_s):
            for c in fire_gathers(g, bset, sem_g):
                c.wait()
            fire_stores(g, bset, sem_s)

        run_group(0, seta, sga, ssa)
        run_group(1, setb, sgb, ssb)

        def body(gp, carry):
            ga = 2 * gp
            drain_stores(ga - 2, seta, ssa)
            run_group(ga, seta, sga, ssa)
            drain_stores(ga - 1, setb, ssb)
            run_group(ga + 1, setb, sgb, ssb)
            return carry

        lax.fori_loop(1, (_NG - 1) // 2, body, 0)
        drain_stores(_NG - 3, seta, ssa)
        run_group(_NG - 1, seta, sga, ssa)
        drain_stores(_NG - 2, setb, ssb)
        drain_stores(_NG - 1, seta, ssa)

    return k(B, dst3)


def _score_topk(A, Bg2, dstN, W2big, REP, b2):
    def body(a_ref, bg_ref, dst_ref, w2_ref, rep_ref, b2_ref,
             src_out, dst_out, w_out):
        i = pl.program_id(0)
        a = a_ref[...]
        bg3 = bg_ref[...]
        arep = lax.dot_general(
            a, rep_ref[...], (((1,), (0,)), ((), ())),
            preferred_element_type=jnp.float32,
            precision=lax.Precision.HIGHEST)
        hidden3 = jnp.maximum(bg3 + arep[:, None, :], 0.0)
        w2big = w2_ref[...]
        logit = b2_ref[0, 0] + jnp.zeros((_NB, _DEG), jnp.float32)
        for r in range(16):
            logit = logit + lax.dot_general(
                hidden3[:, r], w2big[r * 128:(r + 1) * 128],
                (((1,), (0,)), ((), ())),
                preferred_element_type=jnp.float32)
        dstb = dst_ref[...]
        iota = lax.broadcasted_iota(jnp.int32, (_NB, _DEG), 1)
        cur = logit
        sel_dst, sel_w = [], []
        for _ in range(_K):
            m = jnp.max(cur, axis=1, keepdims=True)
            ism = cur == m
            idx = jnp.min(jnp.where(ism, iota, _DEG), axis=1, keepdims=True)
            one = iota == idx
            sel_dst.append(jnp.sum(jnp.where(one, dstb, 0), axis=1, keepdims=True))
            sel_w.append(m)
            cur = jnp.where(one, -jnp.inf, cur)
        nid = i * _NB + lax.broadcasted_iota(jnp.int32, (_NB, _K), 0)
        src_out[...] = nid
        dst_out[...] = jnp.concatenate(sel_dst, axis=1)
        w_out[...] = jax.nn.sigmoid(jnp.concatenate(sel_w, axis=1))

    return pl.pallas_call(
        body,
        grid=(_GRID,),
        in_specs=[
            pl.BlockSpec((_NB, 64), lambda i: (i, 0)),
            pl.BlockSpec((_NB, 16, 128), lambda i: (i, 0, 0)),
            pl.BlockSpec((_NB, _DEG), lambda i: (i, 0)),
            pl.BlockSpec((_D2, _DEG), lambda i: (0, 0)),
            pl.BlockSpec((64, 128), lambda i: (0, 0)),
            pl.BlockSpec((1, 1), lambda i: (0, 0)),
        ],
        out_specs=[
            pl.BlockSpec((_NB, _K), lambda i: (i, 0)),
            pl.BlockSpec((_NB, _K), lambda i: (i, 0)),
            pl.BlockSpec((_NB, _K), lambda i: (i, 0)),
        ],
        out_shape=[
            jax.ShapeDtypeStruct((_N, _K), jnp.int32),
            jax.ShapeDtypeStruct((_N, _K), jnp.int32),
            jax.ShapeDtypeStruct((_N, _K), jnp.float32),
        ],
    )(A, Bg2, dstN, W2big, REP, b2)


def kernel(h, src, dst, W1, b1, W2, b2):
    del src
    W1T = W1.T
    b1r = b1.reshape(1, 64)
    b2r = b2.reshape(1, 1)
    eye = jnp.eye(64, dtype=jnp.float32)
    REP = jnp.tile(eye, (1, 2))
    W2big = jnp.einsum('j,ck->cjk', W2.reshape(64),
                       jnp.eye(_DEG, dtype=jnp.float32)).reshape(_D2, _DEG)
    A, B = _mlp_front(h, W1T, b1r)
    Bg = _sc_gather(B, dst.reshape(_NW, _J, _C))
    src_k, dst_k, w_k = _score_topk(
        A, Bg.reshape(_N, 16, 128), dst.reshape(_N, _DEG), W2big, REP, b2r)
    edge_index = jnp.stack([src_k.reshape(-1), dst_k.reshape(-1)], axis=0)
    edge_w = w_k.reshape(-1)
    return edge_index, edge_w


# speedup vs baseline: 1.1807x; 1.1807x over previous
import functools

import jax
import jax.numpy as jnp
from jax import lax
from jax.experimental import pallas as pl
from jax.experimental.pallas import tpu as pltpu
from jax.experimental.pallas import tpu_sc as plsc

_N = 10000
_DEG = 32
_E = _N * _DEG
_H = 128
_K = 4
_D2 = _DEG * 64

_NC = 2
_NS = 16
_NW = _NC * _NS
_EW = _E // _NW
_C = 80
_J = _EW // _C
_G = 5
_NG = _J // _G

_NB = 2000
_GRID = _N // _NB


def _mlp_front(h, W1T, b1):
    def body(h_ref, w_ref, b1_ref, a_out, b_out):
        hh = h_ref[...]
        w = w_ref[...]
        a_out[...] = lax.dot_general(
            hh, w[:_H], (((1,), (0,)), ((), ())),
            preferred_element_type=jnp.float32) + b1_ref[...]
        b_out[...] = lax.dot_general(
            hh, w[_H:], (((1,), (0,)), ((), ())),
            preferred_element_type=jnp.float32)

    return pl.pallas_call(
        body,
        out_shape=[
            jax.ShapeDtypeStruct((_N, 64), jnp.float32),
            jax.ShapeDtypeStruct((_N, 64), jnp.float32),
        ],
    )(h, W1T, b1)


def _sc_gather(B, dst3):
    mesh = plsc.VectorSubcoreMesh(core_axis_name="c", subcore_axis_name="s")

    @functools.partial(
        pl.kernel,
        out_type=jax.ShapeDtypeStruct((_E, 64), jnp.float32),
        mesh=mesh,
        compiler_params=pltpu.CompilerParams(use_tc_tiling_on_sc=False),
        scratch_types=[
            pltpu.VMEM((_J, _C), jnp.int32),
            [pltpu.VMEM((_C, 64), jnp.float32) for _ in range(2 * _G)],
            pltpu.SemaphoreType.DMA,
            pltpu.SemaphoreType.DMA,
            pltpu.SemaphoreType.DMA,
            pltpu.SemaphoreType.DMA,
        ],
    )
    def k(b_hbm, dst_hbm, out_hbm, idx_v, bufs, sga, sgb, ssa, ssb):
        wid = lax.axis_index("s") * _NC + lax.axis_index("c")
        pltpu.sync_copy(dst_hbm.at[wid], idx_v)
        ebase = wid * _EW
        seta, setb = bufs[:_G], bufs[_G:]

        def fire_gathers(g, bset, sem):
            return [
                pltpu.async_copy(b_hbm.at[idx_v.at[g * _G + b]], bset[b], sem)
                for b in range(_G)
            ]

        def out_slice(g, b):
            off = pl.multiple_of(ebase + (g * _G + b) * _C, 8)
            return out_hbm.at[pl.ds(off, _C)]

        def fire_stores(g, bset, sem):
            for b in range(_G):
                pltpu.async_copy(bset[b], out_slice(g, b), sem)

        def drain_stores(g, bset, sem):
            for b in range(_G):
                pltpu.make_async_copy(bset[b], out_slice(g, b), sem).wait()

        def run_group(g, bset, sem_g, sem_s):
            for c in fire_gathers(g, bset, sem_g):
                c.wait()
            fire_stores(g, bset, sem_s)

        run_group(0, seta, sga, ssa)
        run_group(1, setb, sgb, ssb)

        def body(gp, carry):
            ga = 2 * gp
            drain_stores(ga - 2, seta, ssa)
            run_group(ga, seta, sga, ssa)
            drain_stores(ga - 1, setb, ssb)
            run_group(ga + 1, setb, sgb, ssb)
            return carry

        lax.fori_loop(1, (_NG - 1) // 2, body, 0)
        drain_stores(_NG - 3, seta, ssa)
        run_group(_NG - 1, seta, sga, ssa)
        drain_stores(_NG - 2, setb, ssb)
        drain_stores(_NG - 1, seta, ssa)

    return k(B, dst3)


def _score_topk(A, Bg2, dstN, W2big, REP, b2):
    def body(a_ref, bg_ref, dst_ref, w2_ref, rep_ref, b2_ref,
             src_out, dst_out, w_out):
        i = pl.program_id(0)
        a = a_ref[...]
        bg3 = bg_ref[...]
        arep = lax.dot_general(
            a, rep_ref[...], (((1,), (0,)), ((), ())),
            preferred_element_type=jnp.float32,
            precision=lax.Precision.HIGHEST)
        hidden3 = jnp.maximum(bg3 + arep[:, None, :], 0.0)
        w2big = w2_ref[...]
        logit = b2_ref[0, 0] + jnp.zeros((_NB, _DEG), jnp.float32)
        for r in range(16):
            logit = logit + lax.dot_general(
                hidden3[:, r], w2big[r * 128:(r + 1) * 128],
                (((1,), (0,)), ((), ())),
                preferred_element_type=jnp.float32)
        dstb = dst_ref[...]
        iota = lax.broadcasted_iota(jnp.int32, (_NB, _DEG), 1)
        cur = logit
        sel_dst, sel_w = [], []
        for _ in range(_K):
            m = jnp.max(cur, axis=1, keepdims=True)
            ism = cur == m
            idx = jnp.min(jnp.where(ism, iota, _DEG), axis=1, keepdims=True)
            one = iota == idx
            sel_dst.append(jnp.sum(jnp.where(one, dstb, 0), axis=1, keepdims=True))
            sel_w.append(m)
            cur = jnp.where(one, -jnp.inf, cur)
        nid = i * _NB + lax.broadcasted_iota(jnp.int32, (_NB, _K), 0)
        src_out[...] = nid
        dst_out[...] = jnp.concatenate(sel_dst, axis=1)
        w_out[...] = jax.nn.sigmoid(jnp.concatenate(sel_w, axis=1))

    return pl.pallas_call(
        body,
        grid=(_GRID,),
        in_specs=[
            pl.BlockSpec((_NB, 64), lambda i: (i, 0)),
            pl.BlockSpec((_NB, 16, 128), lambda i: (i, 0, 0)),
            pl.BlockSpec((_NB, _DEG), lambda i: (i, 0)),
            pl.BlockSpec((_D2, _DEG), lambda i: (0, 0)),
            pl.BlockSpec((64, 128), lambda i: (0, 0)),
            pl.BlockSpec((1, 1), lambda i: (0, 0)),
        ],
        out_specs=[
            pl.BlockSpec((_NB, _K), lambda i: (i, 0)),
            pl.BlockSpec((_NB, _K), lambda i: (i, 0)),
            pl.BlockSpec((_NB, _K), lambda i: (i, 0)),
        ],
        out_shape=[
            jax.ShapeDtypeStruct((_N, _K), jnp.int32),
            jax.ShapeDtypeStruct((_N, _K), jnp.int32),
            jax.ShapeDtypeStruct((_N, _K), jnp.float32),
        ],
    )(A, Bg2, dstN, W2big, REP, b2)


def kernel(h, src, dst, W1, b1, W2, b2):
    del src
    W1T = W1.T
    b1r = b1.reshape(1, 64)
    b2r = b2.reshape(1, 1)
    eye = jnp.eye(64, dtype=jnp.float32)
    REP = jnp.tile(eye, (1, 2))
    W2big = jnp.einsum('j,ck->cjk', W2.reshape(64),
                       jnp.eye(_DEG, dtype=jnp.float32)).reshape(_D2, _DEG)
    A, B = _mlp_front(h, W1T, b1r)
    Bg = _sc_gather(B, dst.reshape(_NW, _J, _C))
    src_k, dst_k, w_k = _score_topk(
        A, Bg.reshape(_N, 16, 128), dst.reshape(_N, _DEG), W2big, REP, b2r)
    edge_index = jnp.stack([src_k.reshape(-1), dst_k.reshape(-1)], axis=0)
    edge_w = w_k.reshape(-1)
    return edge_index, edge_w
